# baseline (speedup 1.0000x reference)
import functools

import jax
import jax.numpy as jnp
from jax import lax
from jax.experimental import pallas as pl
from jax.experimental.pallas import tpu as pltpu
from jax.experimental.pallas import tpu_sc as plsc

N = 10000
E = 320000
D = 128
DH = D // 2

NC = 2
NS = 16
CHUNK = 80
EPT = E // NS
NCHUNK = EPT // CHUNK
NP = 10240
RPT = NP // NS
ZROWS = 80
ZDROWS = 128
DEGW = 16


def _sc_aggregate(src2, dst, feat2):
    mesh = plsc.VectorSubcoreMesh(core_axis_name="c", subcore_axis_name="s")

    @functools.partial(
        pl.kernel,
        out_type=[
            jax.ShapeDtypeStruct((NC, NP, DH), jnp.float32),
            jax.ShapeDtypeStruct((NP, DEGW), jnp.float32),
        ],
        mesh=mesh,
        scratch_types=[
            pltpu.VMEM((NCHUNK, CHUNK), jnp.int32),
            pltpu.VMEM((NCHUNK, CHUNK), jnp.int32),
            pltpu.VMEM((CHUNK, DH), jnp.float32),
            pltpu.VMEM((CHUNK, DH), jnp.float32),
            pltpu.VMEM((CHUNK, DH), jnp.float32),
            pltpu.VMEM((CHUNK, DH), jnp.float32),
            pltpu.VMEM((CHUNK, DEGW), jnp.float32),
            pltpu.VMEM((ZROWS, DH), jnp.float32),
            pltpu.VMEM((ZDROWS, DEGW), jnp.float32),
            pltpu.VMEM_SHARED((NP, DH), jnp.float32),
            pltpu.VMEM_SHARED((NP, DEGW), jnp.float32),
            pltpu.SemaphoreType.DMA,
            pltpu.SemaphoreType.DMA,
            pltpu.SemaphoreType.DMA,
        ],
        compiler_params=pltpu.CompilerParams(use_tc_tiling_on_sc=False),
    )
    def k(src_hbm2, dst_hbm, feat_hbm, sum_out, deg_out,
          src_v, dst_v, g0, g1, g2, g3, ones_v, zsum_v, zdeg_v,
          acc_sh, deg_sh, gsem, ssem, osem):
        cid = lax.axis_index("c")
        sid = lax.axis_index("s")
        row0 = sid * RPT
        bufs = [g0, g1, g2, g3]

        pltpu.async_copy(src_hbm2.at[cid, sid], src_v, gsem)
        pltpu.async_copy(dst_hbm.at[sid], dst_v, gsem)

        zero16 = jnp.zeros((16,), jnp.float32)
        one16 = jnp.ones((16,), jnp.float32)

        def fill_ones(i, carry):
            ones_v[i] = one16
            return carry
        lax.fori_loop(0, CHUNK, fill_ones, 0)

        def fill_zsum(i, carry):
            r = i // (DH // 16)
            c = lax.rem(i, DH // 16)
            zsum_v[r, pl.ds(c * 16, 16)] = zero16
            return carry
        lax.fori_loop(0, ZROWS * (DH // 16), fill_zsum, 0)

        def fill_zdeg(i, carry):
            zdeg_v[i] = zero16
            return carry
        lax.fori_loop(0, ZDROWS, fill_zdeg, 0)

        def zacc(t, carry):
            pltpu.sync_copy(zsum_v, acc_sh.at[pl.ds(row0 + t * ZROWS, ZROWS)])
            return carry
        lax.fori_loop(0, RPT // ZROWS, zacc, 0)

        def zdeg(t, carry):
            pltpu.sync_copy(zdeg_v,
                            deg_sh.at[pl.ds(row0 + t * ZDROWS, ZDROWS)])
            return carry
        lax.fori_loop(0, RPT // ZDROWS, zdeg, 0)

        pltpu.make_async_copy(src_hbm2.at[cid, sid], src_v, gsem).wait()
        pltpu.make_async_copy(dst_hbm.at[sid], dst_v, gsem).wait()

        plsc.subcore_barrier()

        def edge_loop(lo):
            feat_ref = feat_hbm
            def gissue(j, buf):
                pltpu.async_copy(feat_ref.at[src_v.at[j]], buf, gsem)

            def gwait(j, buf):
                pltpu.make_async_copy(feat_ref.at[src_v.at[j]],
                                      buf, gsem).wait()

            def sissue(j, buf):
                pltpu.async_copy(buf, acc_sh.at[dst_v.at[j]], ssem, add=True)

            def swait(j, buf):
                pltpu.make_async_copy(buf, acc_sh.at[dst_v.at[j]],
                                      ssem).wait()

            def oissue(j):
                if lo:
                    pltpu.async_copy(ones_v, deg_sh.at[dst_v.at[j]],
                                     osem, add=True)

            def owait(j):
                if lo:
                    pltpu.make_async_copy(ones_v, deg_sh.at[dst_v.at[j]],
                                          osem).wait()

            gissue(0, bufs[0])
            gissue(1, bufs[1])

            def phase(j, p):
                gwait(j, bufs[p])
                sissue(j, bufs[p])
                oissue(j)

                @pl.when(j >= 2)
                def _():
                    swait(j - 2, bufs[(p + 2) % 4])
                    owait(j - 2)

                j2 = jnp.minimum(j + 2, NCHUNK - 1)
                gissue(j2, bufs[(p + 2) % 4])

            def body(jj, carry):
                j0 = 4 * jj
                phase(j0, 0)
                phase(j0 + 1, 1)
                phase(j0 + 2, 2)
                phase(j0 + 3, 3)
                return carry
            lax.fori_loop(0, NCHUNK // 4, body, 0)
            phase(jnp.int32(NCHUNK - 2), (NCHUNK - 2) % 4)
            phase(jnp.int32(NCHUNK - 1), (NCHUNK - 1) % 4)

            swait(NCHUNK - 2, bufs[(NCHUNK - 2) % 4])
            owait(NCHUNK - 2)
            swait(NCHUNK - 1, bufs[(NCHUNK - 1) % 4])
            owait(NCHUNK - 1)
            gwait(NCHUNK - 1, bufs[NCHUNK % 4])
            gwait(NCHUNK - 1, bufs[(NCHUNK + 1) % 4])

        @pl.when(cid == 0)
        def _():
            edge_loop(True)

        @pl.when(cid == 1)
        def _():
            edge_loop(False)

        plsc.subcore_barrier()

        pltpu.sync_copy(acc_sh.at[pl.ds(row0, RPT)],
                        sum_out.at[cid, pl.ds(row0, RPT)])
        @pl.when(cid == 0)
        def _():
            pltpu.sync_copy(deg_sh.at[pl.ds(row0, RPT)],
                            deg_out.at[pl.ds(row0, RPT)])

    return k(src2, dst, feat2)


def _tc_finalize(sums, degs, W):
    R = 1000

    def body(s_ref, d_ref, w_ref, o_ref):
        s = jnp.concatenate([s_ref[0], s_ref[1]], axis=1)
        deg = d_ref[:, :1]
        rst = s / jnp.maximum(deg, 1.0)
        out = jnp.dot(rst, w_ref[...], preferred_element_type=jnp.float32)
        o_ref[...] = jnp.maximum(out, 0.0)

    return pl.pallas_call(
        body,
        grid=(N // R,),
        in_specs=[
            pl.BlockSpec((NC, R, DH), lambda i: (0, i, 0)),
            pl.BlockSpec((R, DEGW), lambda i: (i, 0)),
            pl.BlockSpec((D, D), lambda i: (0, 0)),
        ],
        out_specs=pl.BlockSpec((R, D), lambda i: (i, 0)),
        out_shape=jax.ShapeDtypeStruct((N, D), jnp.float32),
    )(sums, degs, W)


@jax.jit
def kernel(feat, edge_index, W):
    src = edge_index[0]
    src2 = jnp.stack([src * 2, src * 2 + 1]).reshape(NC, NS, NCHUNK, CHUNK)
    dst = edge_index[1].reshape(NS, NCHUNK, CHUNK)
    feat2 = feat.reshape(2 * N, DH)
    sums, degs = _sc_aggregate(src2, dst, feat2)
    return _tc_finalize(sums, degs, W)


# speedup vs baseline: 1.0328x; 1.0328x over previous
import functools

import jax
import jax.numpy as jnp
from jax import lax
from jax.experimental import pallas as pl
from jax.experimental.pallas import tpu as pltpu
from jax.experimental.pallas import tpu_sc as plsc

N = 10000
E = 320000
D = 128
DH = D // 2

NC = 2
NS = 16
CHUNK = 120
NCHUNK = 167
EPT = NCHUNK * CHUNK
EPAD = NS * EPT - E
TAIL = NCHUNK % 4
NP = 10240
JUNK = NP - 1
RPT = NP // NS
ZROWS = 80
ZDROWS = 64
DEGW = 16


def _sc_aggregate(ei, feat_cols):
    mesh = plsc.VectorSubcoreMesh(core_axis_name="c", subcore_axis_name="s")

    @functools.partial(
        pl.kernel,
        out_type=[
            jax.ShapeDtypeStruct((NC, NP, DH), jnp.float32),
            jax.ShapeDtypeStruct((NP, DEGW), jnp.float32),
        ],
        mesh=mesh,
        scratch_types=[
            pltpu.VMEM((NCHUNK, CHUNK), jnp.int32),
            pltpu.VMEM((NCHUNK, CHUNK), jnp.int32),
            pltpu.VMEM((CHUNK, DH), jnp.float32),
            pltpu.VMEM((CHUNK, DH), jnp.float32),
            pltpu.VMEM((CHUNK, DH), jnp.float32),
            pltpu.VMEM((CHUNK, DH), jnp.float32),
            pltpu.VMEM((CHUNK, DEGW), jnp.float32),
            pltpu.VMEM((ZROWS, DH), jnp.float32),
            pltpu.VMEM((ZDROWS, DEGW), jnp.float32),
            pltpu.VMEM_SHARED((NP, DH), jnp.float32),
            pltpu.VMEM_SHARED((NP, DEGW), jnp.float32),
            pltpu.SemaphoreType.DMA,
            pltpu.SemaphoreType.DMA,
            pltpu.SemaphoreType.DMA,
        ],
        compiler_params=pltpu.CompilerParams(use_tc_tiling_on_sc=False),
    )
    def k(ei_hbm, feat_hbm, sum_out, deg_out,
          src_v, dst_v, g0, g1, g2, g3, ones_v, zsum_v, zdeg_v,
          acc_sh, deg_sh, gsem, ssem, osem):
        cid = lax.axis_index("c")
        sid = lax.axis_index("s")
        row0 = sid * RPT
        bufs = [g0, g1, g2, g3]

        pltpu.async_copy(ei_hbm.at[0, sid], src_v, gsem)
        pltpu.async_copy(ei_hbm.at[1, sid], dst_v, gsem)

        zero16 = jnp.zeros((16,), jnp.float32)
        one16 = jnp.ones((16,), jnp.float32)

        def fill_ones(i, carry):
            ones_v[i] = one16
            return carry
        lax.fori_loop(0, CHUNK, fill_ones, 0)

        def fill_zsum(i, carry):
            r = i // (DH // 16)
            c = lax.rem(i, DH // 16)
            zsum_v[r, pl.ds(c * 16, 16)] = zero16
            return carry
        lax.fori_loop(0, ZROWS * (DH // 16), fill_zsum, 0)

        def fill_zdeg(i, carry):
            zdeg_v[i] = zero16
            return carry
        lax.fori_loop(0, ZDROWS, fill_zdeg, 0)

        def zacc(t, carry):
            pltpu.sync_copy(zsum_v, acc_sh.at[pl.ds(row0 + t * ZROWS, ZROWS)])
            return carry
        lax.fori_loop(0, RPT // ZROWS, zacc, 0)

        def zdeg(t, carry):
            pltpu.sync_copy(zdeg_v,
                            deg_sh.at[pl.ds(row0 + t * ZDROWS, ZDROWS)])
            return carry
        lax.fori_loop(0, RPT // ZDROWS, zdeg, 0)

        pltpu.make_async_copy(ei_hbm.at[0, sid], src_v, gsem).wait()
        pltpu.make_async_copy(ei_hbm.at[1, sid], dst_v, gsem).wait()

        plsc.subcore_barrier()

        def edge_loop(feat_ref, lo):
            def gissue(j, buf):
                pltpu.async_copy(feat_ref.at[src_v.at[j]], buf, gsem)

            def gwait(j, buf):
                pltpu.make_async_copy(feat_ref.at[src_v.at[j]],
                                      buf, gsem).wait()

            def sissue(j, buf):
                pltpu.async_copy(buf, acc_sh.at[dst_v.at[j]], ssem, add=True)

            def swait(j, buf):
                pltpu.make_async_copy(buf, acc_sh.at[dst_v.at[j]],
                                      ssem).wait()

            def oissue(j):
                if lo:
                    pltpu.async_copy(ones_v, deg_sh.at[dst_v.at[j]],
                                     osem, add=True)

            def owait(j):
                if lo:
                    pltpu.make_async_copy(ones_v, deg_sh.at[dst_v.at[j]],
                                          osem).wait()

            gissue(0, bufs[0])
            gissue(1, bufs[1])

            def phase(j, p):
                gwait(j, bufs[p])
                sissue(j, bufs[p])
                oissue(j)

                @pl.when(j >= 2)
                def _():
                    swait(j - 2, bufs[(p + 2) % 4])
                    owait(j - 2)

                j2 = jnp.minimum(j + 2, NCHUNK - 1)
                gissue(j2, bufs[(p + 2) % 4])

            def body(jj, carry):
                j0 = 4 * jj
                phase(j0, 0)
                phase(j0 + 1, 1)
                phase(j0 + 2, 2)
                phase(j0 + 3, 3)
                return carry
            lax.fori_loop(0, NCHUNK // 4, body, 0)
            for j in range(NCHUNK - TAIL, NCHUNK):
                phase(jnp.int32(j), j % 4)

            swait(NCHUNK - 2, bufs[(NCHUNK - 2) % 4])
            owait(NCHUNK - 2)
            swait(NCHUNK - 1, bufs[(NCHUNK - 1) % 4])
            owait(NCHUNK - 1)
            gwait(NCHUNK - 1, bufs[NCHUNK % 4])
            gwait(NCHUNK - 1, bufs[(NCHUNK + 1) % 4])

        @pl.when(cid == 0)
        def _():
            edge_loop(feat_hbm.at[0], True)

        @pl.when(cid == 1)
        def _():
            edge_loop(feat_hbm.at[1], False)

        plsc.subcore_barrier()

        pltpu.sync_copy(acc_sh.at[pl.ds(row0, RPT)],
                        sum_out.at[cid, pl.ds(row0, RPT)])
        @pl.when(cid == 0)
        def _():
            pltpu.sync_copy(deg_sh.at[pl.ds(row0, RPT)],
                            deg_out.at[pl.ds(row0, RPT)])

    return k(ei, feat_cols)


def _tc_finalize(sums, degs, W):
    R = 1000

    def body(s_ref, d_ref, w_ref, o_ref):
        s = jnp.concatenate([s_ref[0], s_ref[1]], axis=1)
        deg = d_ref[:, :1]
        rst = s / jnp.maximum(deg, 1.0)
        out = jnp.dot(rst, w_ref[...], preferred_element_type=jnp.float32)
        o_ref[...] = jnp.maximum(out, 0.0)

    return pl.pallas_call(
        body,
        grid=(N // R,),
        in_specs=[
            pl.BlockSpec((NC, R, DH), lambda i: (0, i, 0)),
            pl.BlockSpec((R, DEGW), lambda i: (i, 0)),
            pl.BlockSpec((D, D), lambda i: (0, 0)),
        ],
        out_specs=pl.BlockSpec((R, D), lambda i: (i, 0)),
        out_shape=jax.ShapeDtypeStruct((N, D), jnp.float32),
    )(sums, degs, W)


@jax.jit
def kernel(feat, edge_index, W):
    pad = jnp.concatenate(
        [jnp.zeros((1, EPAD), jnp.int32),
         jnp.full((1, EPAD), JUNK, jnp.int32)], axis=0)
    ei = jnp.concatenate([edge_index, pad], axis=1)
    ei = ei.reshape(2, NS, NCHUNK, CHUNK)
    feat_cols = jnp.stack([feat[:, :DH], feat[:, DH:]])
    sums, degs = _sc_aggregate(ei, feat_cols)
    return _tc_finalize(sums, degs, W)


# speedup vs baseline: 1.0878x; 1.0533x over previous
import functools

import jax
import jax.numpy as jnp
from jax import lax
from jax.experimental import pallas as pl
from jax.experimental.pallas import tpu as pltpu
from jax.experimental.pallas import tpu_sc as plsc

N = 10000
E = 320000
D = 128
DH = D // 2

NC = 2
NS = 16
CHUNK = 80
EPT = E // NS
NCHUNK = EPT // CHUNK
NP = 10240
RPT = NP // NS
ZROWS = 80
ZDROWS = 128
DEGW = 16


def _sc_aggregate(ei, feat_cols):
    mesh = plsc.VectorSubcoreMesh(core_axis_name="c", subcore_axis_name="s")

    @functools.partial(
        pl.kernel,
        out_type=[
            jax.ShapeDtypeStruct((NP, D), jnp.float32),
            jax.ShapeDtypeStruct((NP, DEGW), jnp.float32),
        ],
        mesh=mesh,
        scratch_types=[
            pltpu.VMEM((NCHUNK, CHUNK), jnp.int32),
            pltpu.VMEM((NCHUNK, CHUNK), jnp.int32),
            pltpu.VMEM((CHUNK, DH), jnp.float32),
            pltpu.VMEM((CHUNK, DH), jnp.float32),
            pltpu.VMEM((CHUNK, DH), jnp.float32),
            pltpu.VMEM((CHUNK, DH), jnp.float32),
            pltpu.VMEM((CHUNK, DEGW), jnp.float32),
            pltpu.VMEM((ZROWS, DH), jnp.float32),
            pltpu.VMEM((ZDROWS, DEGW), jnp.float32),
            pltpu.VMEM_SHARED((NP, DH), jnp.float32),
            pltpu.VMEM_SHARED((NP, DEGW), jnp.float32),
            pltpu.SemaphoreType.DMA,
            pltpu.SemaphoreType.DMA,
            pltpu.SemaphoreType.DMA,
        ],
        compiler_params=pltpu.CompilerParams(use_tc_tiling_on_sc=False),
    )
    def k(ei_hbm, feat_hbm, sum_out, deg_out,
          src_v, dst_v, g0, g1, g2, g3, ones_v, zsum_v, zdeg_v,
          acc_sh, deg_sh, gsem, ssem, osem):
        cid = lax.axis_index("c")
        sid = lax.axis_index("s")
        row0 = sid * RPT
        bufs = [g0, g1, g2, g3]

        pltpu.async_copy(ei_hbm.at[0, sid], src_v, gsem)
        pltpu.async_copy(ei_hbm.at[1, sid], dst_v, gsem)

        zero16 = jnp.zeros((16,), jnp.float32)
        one16 = jnp.ones((16,), jnp.float32)

        def fill_ones(i, carry):
            ones_v[i] = one16
            return carry
        lax.fori_loop(0, CHUNK, fill_ones, 0)

        def fill_zsum(i, carry):
            r = i // (DH // 16)
            c = lax.rem(i, DH // 16)
            zsum_v[r, pl.ds(c * 16, 16)] = zero16
            return carry
        lax.fori_loop(0, ZROWS * (DH // 16), fill_zsum, 0)

        def fill_zdeg(i, carry):
            zdeg_v[i] = zero16
            return carry
        lax.fori_loop(0, ZDROWS, fill_zdeg, 0)

        def zacc(t, carry):
            pltpu.sync_copy(zsum_v, acc_sh.at[pl.ds(row0 + t * ZROWS, ZROWS)])
            return carry
        lax.fori_loop(0, RPT // ZROWS, zacc, 0)

        def zdeg(t, carry):
            pltpu.sync_copy(zdeg_v,
                            deg_sh.at[pl.ds(row0 + t * ZDROWS, ZDROWS)])
            return carry
        lax.fori_loop(0, RPT // ZDROWS, zdeg, 0)

        pltpu.make_async_copy(ei_hbm.at[0, sid], src_v, gsem).wait()
        pltpu.make_async_copy(ei_hbm.at[1, sid], dst_v, gsem).wait()

        plsc.subcore_barrier()

        def edge_loop(feat_ref, lo):
            def gissue(j, buf):
                pltpu.async_copy(feat_ref.at[src_v.at[j]], buf, gsem)

            def gwait(j, buf):
                pltpu.make_async_copy(feat_ref.at[src_v.at[j]],
                                      buf, gsem).wait()

            def sissue(j, buf):
                pltpu.async_copy(buf, acc_sh.at[dst_v.at[j]], ssem, add=True)

            def swait(j, buf):
                pltpu.make_async_copy(buf, acc_sh.at[dst_v.at[j]],
                                      ssem).wait()

            def oissue(j):
                if lo:
                    pltpu.async_copy(ones_v, deg_sh.at[dst_v.at[j]],
                                     osem, add=True)

            def owait(j):
                if lo:
                    pltpu.make_async_copy(ones_v, deg_sh.at[dst_v.at[j]],
                                          osem).wait()

            gissue(0, bufs[0])
            gissue(1, bufs[1])

            def phase(j, p):
                gwait(j, bufs[p])
                sissue(j, bufs[p])
                oissue(j)

                @pl.when(j >= 2)
                def _():
                    swait(j - 2, bufs[(p + 2) % 4])
                    owait(j - 2)

                j2 = jnp.minimum(j + 2, NCHUNK - 1)
                gissue(j2, bufs[(p + 2) % 4])

            def body(jj, carry):
                j0 = 4 * jj
                phase(j0, 0)
                phase(j0 + 1, 1)
                phase(j0 + 2, 2)
                phase(j0 + 3, 3)
                return carry
            lax.fori_loop(0, NCHUNK // 4, body, 0)
            phase(jnp.int32(NCHUNK - 2), (NCHUNK - 2) % 4)
            phase(jnp.int32(NCHUNK - 1), (NCHUNK - 1) % 4)

            swait(NCHUNK - 2, bufs[(NCHUNK - 2) % 4])
            owait(NCHUNK - 2)
            swait(NCHUNK - 1, bufs[(NCHUNK - 1) % 4])
            owait(NCHUNK - 1)
            gwait(NCHUNK - 1, bufs[NCHUNK % 4])
            gwait(NCHUNK - 1, bufs[(NCHUNK + 1) % 4])

        @pl.when(cid == 0)
        def _():
            edge_loop(feat_hbm.at[0], True)

        @pl.when(cid == 1)
        def _():
            edge_loop(feat_hbm.at[1], False)

        plsc.subcore_barrier()

        @pl.when(cid == 0)
        def _():
            pltpu.sync_copy(acc_sh.at[pl.ds(row0, RPT)],
                            sum_out.at[pl.ds(row0, RPT), pl.ds(0, DH)])
            pltpu.sync_copy(deg_sh.at[pl.ds(row0, RPT)],
                            deg_out.at[pl.ds(row0, RPT)])

        @pl.when(cid == 1)
        def _():
            pltpu.sync_copy(acc_sh.at[pl.ds(row0, RPT)],
                            sum_out.at[pl.ds(row0, RPT), pl.ds(DH, DH)])

    return k(ei, feat_cols)


def _tc_finalize(sums, degs, W):
    R = 1000

    def body(s_ref, d_ref, w_ref, o_ref):
        s = s_ref[...]
        deg = d_ref[:, :1]
        rst = s / jnp.maximum(deg, 1.0)
        out = jnp.dot(rst, w_ref[...], preferred_element_type=jnp.float32)
        o_ref[...] = jnp.maximum(out, 0.0)

    return pl.pallas_call(
        body,
        grid=(N // R,),
        in_specs=[
            pl.BlockSpec((R, D), lambda i: (i, 0)),
            pl.BlockSpec((R, DEGW), lambda i: (i, 0)),
            pl.BlockSpec((D, D), lambda i: (0, 0)),
        ],
        out_specs=pl.BlockSpec((R, D), lambda i: (i, 0)),
        out_shape=jax.ShapeDtypeStruct((N, D), jnp.float32),
    )(sums, degs, W)


@jax.jit
def kernel(feat, edge_index, W):
    ei = edge_index.reshape(2, NS, NCHUNK, CHUNK)
    feat_cols = jnp.stack([feat[:, :DH], feat[:, DH:]])
    sums, degs = _sc_aggregate(ei, feat_cols)
    return _tc_finalize(sums, degs, W)
